# baseline (speedup 1.0000x reference)
import functools
import jax
import jax.numpy as jnp
from jax import lax
from jax.experimental import pallas as pl
from jax.experimental.pallas import tpu as pltpu
from jax.experimental.pallas import tpu_sc as plsc

E = 8
D = 768
T = 256


def _sc_gather_rows(src, idx, B):
    info = plsc.get_sparse_core_info()
    NC, NS = info.num_cores, info.num_subcores
    NW = NC * NS
    b_per_w = B // NW
    mesh = plsc.VectorSubcoreMesh(core_axis_name="c", subcore_axis_name="s")

    @functools.partial(
        pl.kernel, mesh=mesh,
        out_type=jax.ShapeDtypeStruct((B, D), jnp.float32),
        scratch_types=[
            pltpu.VMEM((b_per_w,), jnp.int32),
            pltpu.VMEM((b_per_w, D), jnp.float32),
            pltpu.SemaphoreType.DMA,
        ],
    )
    def gather_rows(src_hbm, idx_hbm, out_hbm, idx_v, rows_v, sem):
        wid = lax.axis_index("s") * NC + lax.axis_index("c")
        base = wid * b_per_w
        pltpu.sync_copy(idx_hbm.at[pl.ds(base, b_per_w)], idx_v)
        pltpu.async_copy(src_hbm.at[idx_v], rows_v, sem).wait()
        pltpu.sync_copy(rows_v, out_hbm.at[pl.ds(base, b_per_w)])

    return gather_rows(src, idx)


def _grouped_body(sched_ref, xs_ref, W_ref, b_ref, out_ref):
    g = pl.program_id(0)
    tile = sched_ref[0, g]
    start = sched_ref[2, g]
    end = sched_ref[3, g]
    rows = tile * T + lax.broadcasted_iota(jnp.int32, (T, 1), 0)
    mask = jnp.logical_and(rows >= start, rows < end)
    xm = jnp.where(mask, xs_ref[...], 0.0)
    contrib = jnp.dot(xm, W_ref[0], preferred_element_type=jnp.float32)
    contrib = contrib + jnp.where(mask, b_ref[0], 0.0)
    prev_tile = sched_ref[0, jnp.maximum(g - 1, 0)]
    first = jnp.logical_or(g == 0, tile != prev_tile)

    @pl.when(first)
    def _():
        out_ref[...] = contrib

    @pl.when(jnp.logical_not(first))
    def _():
        out_ref[...] = out_ref[...] + contrib


def _schedule(eid, B):
    num_tiles = B // T
    G = num_tiles + E - 1
    counts = jnp.sum(jax.nn.one_hot(eid, E, dtype=jnp.int32), axis=0)
    off = jnp.concatenate([jnp.zeros((1,), jnp.int32), jnp.cumsum(counts)])
    s_e = off[:E] // T
    q_e = jnp.maximum(off[1:] - 1, 0) // T
    n_e = jnp.where(counts > 0, q_e - s_e + 1, 0)
    cn = jnp.cumsum(n_e)
    beta = cn - n_e
    g = jnp.arange(G, dtype=jnp.int32)
    e_g = jnp.searchsorted(cn, g, side="right").astype(jnp.int32)
    P = cn[-1]
    valid = g < P
    e_gc = jnp.clip(e_g, 0, E - 1)
    tile_g = s_e[e_gc] + (g - beta[e_gc])
    last_e = jnp.max(jnp.where(valid, e_gc, 0))
    start_g = jnp.where(valid, jnp.maximum(off[e_gc], tile_g * T), 0)
    end_g = jnp.where(valid, jnp.minimum(off[e_gc + 1], (tile_g + 1) * T), 0)
    tile_g = jnp.where(valid, tile_g, num_tiles - 1)
    expert_g = jnp.where(valid, e_gc, last_e)
    return jnp.stack([tile_g, expert_g, start_g, end_g]).astype(jnp.int32), G


def kernel(x, curr_video_id, W, b):
    B = x.shape[0]
    eid = curr_video_id.astype(jnp.int32)
    perm = jnp.argsort(eid, stable=True).astype(jnp.int32)
    inv = jnp.zeros((B,), jnp.int32).at[perm].set(jnp.arange(B, dtype=jnp.int32))
    sched, G = _schedule(eid, B)

    xs = _sc_gather_rows(x, perm, B)

    b3 = b.reshape(E, 1, D)
    grid_spec = pltpu.PrefetchScalarGridSpec(
        num_scalar_prefetch=1,
        grid=(G,),
        in_specs=[
            pl.BlockSpec((T, D), lambda g, s: (s[0, g], 0)),
            pl.BlockSpec((1, D, D), lambda g, s: (s[1, g], 0, 0)),
            pl.BlockSpec((1, 1, D), lambda g, s: (s[1, g], 0, 0)),
        ],
        out_specs=pl.BlockSpec((T, D), lambda g, s: (s[0, g], 0)),
    )
    ys = pl.pallas_call(
        _grouped_body,
        grid_spec=grid_spec,
        out_shape=jax.ShapeDtypeStruct((B, D), jnp.float32),
    )(sched, xs, W, b3)

    return _sc_gather_rows(ys, inv, B)


# speedup vs baseline: 1.0044x; 1.0044x over previous
import functools
import jax
import jax.numpy as jnp
from jax import lax
from jax.experimental import pallas as pl
from jax.experimental.pallas import tpu as pltpu
from jax.experimental.pallas import tpu_sc as plsc

E = 8
D = 768
T = 256


def _sc_gather_rows(src, idx, B):
    info = plsc.get_sparse_core_info()
    NC, NS = info.num_cores, info.num_subcores
    NW = NC * NS
    b_per_w = B // NW
    mesh = plsc.VectorSubcoreMesh(core_axis_name="c", subcore_axis_name="s")

    @functools.partial(
        pl.kernel, mesh=mesh,
        out_type=jax.ShapeDtypeStruct((B, D), jnp.float32),
        scratch_types=[
            pltpu.VMEM((b_per_w,), jnp.int32),
            pltpu.VMEM((b_per_w, D), jnp.float32),
            pltpu.SemaphoreType.DMA,
        ],
    )
    def gather_rows(src_hbm, idx_hbm, out_hbm, idx_v, rows_v, sem):
        wid = lax.axis_index("s") * NC + lax.axis_index("c")
        base = wid * b_per_w
        pltpu.sync_copy(idx_hbm.at[pl.ds(base, b_per_w)], idx_v)
        pltpu.async_copy(src_hbm.at[idx_v], rows_v, sem).wait()
        pltpu.sync_copy(rows_v, out_hbm.at[pl.ds(base, b_per_w)])

    return gather_rows(src, idx)


def _grouped_body(sched_ref, xs_ref, W_ref, b_ref, out_ref):
    g = pl.program_id(0)
    tile = sched_ref[0, g]
    start = sched_ref[2, g]
    end = sched_ref[3, g]
    rows = tile * T + lax.broadcasted_iota(jnp.int32, (T, 1), 0)
    mask = jnp.logical_and(rows >= start, rows < end)
    raw = jnp.dot(xs_ref[...], W_ref[0], preferred_element_type=jnp.float32)
    contrib = jnp.where(mask, raw + b_ref[0], 0.0)
    prev_tile = sched_ref[0, jnp.maximum(g - 1, 0)]
    first = jnp.logical_or(g == 0, tile != prev_tile)

    @pl.when(first)
    def _():
        out_ref[...] = contrib

    @pl.when(jnp.logical_not(first))
    def _():
        out_ref[...] = out_ref[...] + contrib


def _schedule(off, counts, num_tiles, G):
    s_e = off[:E] // T
    q_e = jnp.maximum(off[1:] - 1, 0) // T
    n_e = jnp.where(counts > 0, q_e - s_e + 1, 0)
    cn = jnp.cumsum(n_e)
    beta = cn - n_e
    g = jnp.arange(G, dtype=jnp.int32)
    e_g = jnp.searchsorted(cn, g, side="right").astype(jnp.int32)
    P = cn[-1]
    valid = g < P
    e_gc = jnp.clip(e_g, 0, E - 1)
    tile_g = s_e[e_gc] + (g - beta[e_gc])
    last_e = jnp.max(jnp.where(valid, e_gc, 0))
    start_g = jnp.where(valid, jnp.maximum(off[e_gc], tile_g * T), 0)
    end_g = jnp.where(valid, jnp.minimum(off[e_gc + 1], (tile_g + 1) * T), 0)
    tile_g = jnp.where(valid, tile_g, num_tiles - 1)
    expert_g = jnp.where(valid, e_gc, last_e)
    return jnp.stack([tile_g, expert_g, start_g, end_g]).astype(jnp.int32)


def kernel(x, curr_video_id, W, b):
    B = x.shape[0]
    num_tiles = B // T
    G = num_tiles + E - 1
    eid = curr_video_id.astype(jnp.int32)

    oh = (eid[:, None] == jnp.arange(E, dtype=jnp.int32)[None, :]).astype(jnp.int32)
    rank = jnp.cumsum(oh, axis=0)
    counts = rank[-1]
    off = jnp.concatenate([jnp.zeros((1,), jnp.int32), jnp.cumsum(counts)])
    pos = jnp.sum(oh * (off[:E][None, :] + rank - 1), axis=1).astype(jnp.int32)
    perm = jnp.zeros((B,), jnp.int32).at[pos].set(jnp.arange(B, dtype=jnp.int32))
    sched = _schedule(off, counts, num_tiles, G)

    xs = _sc_gather_rows(x, perm, B)

    b3 = b.reshape(E, 1, D)
    grid_spec = pltpu.PrefetchScalarGridSpec(
        num_scalar_prefetch=1,
        grid=(G,),
        in_specs=[
            pl.BlockSpec((T, D), lambda g, s: (s[0, g], 0)),
            pl.BlockSpec((1, D, D), lambda g, s: (s[1, g], 0, 0)),
            pl.BlockSpec((1, 1, D), lambda g, s: (s[1, g], 0, 0)),
        ],
        out_specs=pl.BlockSpec((T, D), lambda g, s: (s[0, g], 0)),
    )
    ys = pl.pallas_call(
        _grouped_body,
        grid_spec=grid_spec,
        out_shape=jax.ShapeDtypeStruct((B, D), jnp.float32),
    )(sched, xs, W, b3)

    return _sc_gather_rows(ys, pos, B)


# speedup vs baseline: 2.0494x; 2.0405x over previous
import jax
import jax.numpy as jnp
from jax.experimental import pallas as pl
from jax.experimental.pallas import tpu as pltpu

E = 8
D = 768
T = 256


def _moe_body(onehot_ref, x_ref, W_ref, b_ref, out_ref, wbf_ref):
    g = pl.program_id(0)

    @pl.when(g == 0)
    def _():
        wbf_ref[...] = W_ref[...].astype(jnp.bfloat16)

    oh = onehot_ref[...]
    acc = jnp.dot(oh, b_ref[...], preferred_element_type=jnp.float32)
    xb = x_ref[...].astype(jnp.bfloat16)
    ohb = oh.astype(jnp.bfloat16)
    for e in range(E):
        xm = xb * ohb[:, e:e + 1]
        acc = acc + jnp.dot(xm, wbf_ref[e], preferred_element_type=jnp.float32)
    out_ref[...] = acc


def kernel(x, curr_video_id, W, b):
    B = x.shape[0]
    eid = curr_video_id.astype(jnp.int32)
    onehot = jax.nn.one_hot(eid, E, dtype=x.dtype)
    num_tiles = B // T

    return pl.pallas_call(
        _moe_body,
        grid=(num_tiles,),
        in_specs=[
            pl.BlockSpec((T, E), lambda t: (t, 0)),
            pl.BlockSpec((T, D), lambda t: (t, 0)),
            pl.BlockSpec((E, D, D), lambda t: (0, 0, 0)),
            pl.BlockSpec((E, D), lambda t: (0, 0)),
        ],
        out_specs=pl.BlockSpec((T, D), lambda t: (t, 0)),
        out_shape=jax.ShapeDtypeStruct((B, D), x.dtype),
        scratch_shapes=[pltpu.VMEM((E, D, D), jnp.bfloat16)],
    )(onehot, x, W, b)
